# baseline (speedup 1.0000x reference)
import functools

import jax
import jax.numpy as jnp
from jax import lax
from jax.experimental import pallas as pl
from jax.experimental.pallas import tpu as pltpu
from jax.experimental.pallas import tpu_sc as plsc

B, N, AXIS, NF, K, H = 10, 1000, 3, 128, 16, 128
MAX_DELTA = 0.4
NTOT = B * N
NPAD = 10240
NCOLPAD = 1024
FIN = AXIS + NF
FPAD = 256
INV_DEG = 1.0 / float(K + 1)

SC_NC = 2
SC_NS = 16
CHUNK_NODES = 8
VREGS_PER_ROW = H // 16
NBUF = 4
C0_PAIRS, C1_PAIRS = 26, 14
C0_ROWS = C0_PAIRS * 2 * CHUNK_NODES
C1_ROWS = C1_PAIRS * 2 * CHUNK_NODES
C1_BASE = SC_NS * C0_ROWS
IDX_MAX = C0_ROWS * K
IDX_PAD = (C1_BASE + SC_NS * C1_ROWS - C1_ROWS) * K + IDX_MAX


KNN_R = 200


def _knn_body(pos_row_ref, pos_col_ref, idx_ref):
    b = pl.program_id(0)
    pr = pos_row_ref[0]
    pc = pos_col_ref[0]
    dx = pr[:, 0:1] - pc[0:1, :]
    dy = pr[:, 1:2] - pc[1:2, :]
    dz = pr[:, 2:3] - pc[2:3, :]
    d2 = (dx * dx + dy * dy) + dz * dz
    lane = lax.broadcasted_iota(jnp.int32, (KNN_R, NCOLPAD), 1
                                ).astype(jnp.float32)
    d2 = jnp.where(lane >= float(N), jnp.inf, d2)
    col16 = lax.broadcasted_iota(jnp.int32, (KNN_R, K), 1)
    acc = jnp.zeros((KNN_R, K), jnp.int32)
    big = jnp.float32(3e9)
    for t in range(K):
        m = jnp.min(d2, axis=1, keepdims=True)
        cand = jnp.where(d2 == m, lane, big)
        sel = jnp.min(cand, axis=1, keepdims=True)
        acc = jnp.where(col16 == t, sel.astype(jnp.int32), acc)
        d2 = jnp.where(lane == sel, jnp.inf, d2)
    idx_ref[0] = acc + b * N


def _knn_indices(pos):
    pos_row = jnp.pad(pos, ((0, 0), (0, NCOLPAD - N), (0, 8 - AXIS)))
    pos_col = jnp.pad(jnp.transpose(pos, (0, 2, 1)),
                      ((0, 0), (0, 8 - AXIS), (0, NCOLPAD - N)))
    grid = (B, N // KNN_R)
    return pl.pallas_call(
        _knn_body,
        grid=grid,
        in_specs=[
            pl.BlockSpec((1, KNN_R, 8), lambda b, r: (b, r, 0)),
            pl.BlockSpec((1, 8, NCOLPAD), lambda b, r: (b, 0, 0)),
        ],
        out_specs=pl.BlockSpec((1, KNN_R, K), lambda b, r: (b, r, 0)),
        out_shape=jax.ShapeDtypeStruct((B, N, K), jnp.int32),
    )(pos_row, pos_col)


MM_R = 512


def _h1_body(x_ref, w_ref, o_ref):
    o_ref[...] = jnp.dot(x_ref[...], w_ref[...],
                         preferred_element_type=jnp.float32)


def _mm_h1(xpad, w1pad):
    return pl.pallas_call(
        _h1_body,
        grid=(NPAD // MM_R,),
        in_specs=[
            pl.BlockSpec((MM_R, FPAD), lambda i: (i, 0)),
            pl.BlockSpec((FPAD, H), lambda i: (0, 0)),
        ],
        out_specs=pl.BlockSpec((MM_R, H), lambda i: (i, 0)),
        out_shape=jax.ShapeDtypeStruct((NPAD, H), jnp.float32),
    )(xpad, w1pad)


def _mlp_body(s1_ref, h1_ref, b1_ref, wm1_ref, bm1_ref, wm2_ref, bm2_ref, o_ref):
    x1 = jax.nn.relu((s1_ref[...] + h1_ref[...]) * INV_DEG + b1_ref[...])
    t = jax.nn.relu(jnp.dot(x1, wm1_ref[...],
                            preferred_element_type=jnp.float32) + bm1_ref[...])
    o_ref[...] = jnp.dot(t, wm2_ref[...],
                         preferred_element_type=jnp.float32) + bm2_ref[...]


def _mm_mlp(s1, h1, b1, wm1, bm1, wm2, bm2):
    full = lambda shape: pl.BlockSpec(shape, lambda i: (0, 0))
    return pl.pallas_call(
        _mlp_body,
        grid=(NPAD // MM_R,),
        in_specs=[
            pl.BlockSpec((MM_R, H), lambda i: (i, 0)),
            pl.BlockSpec((MM_R, H), lambda i: (i, 0)),
            full((1, H)), full((H, H)), full((1, H)), full((H, H)), full((1, H)),
        ],
        out_specs=pl.BlockSpec((MM_R, H), lambda i: (i, 0)),
        out_shape=jax.ShapeDtypeStruct((NPAD, H), jnp.float32),
    )(s1, h1, b1.reshape(1, H), wm1, bm1.reshape(1, H), wm2, bm2.reshape(1, H))


def _out_body(s2_ref, x2_ref, w2_ref, b2_ref, scale_ref, o_ref):
    g = jnp.dot((s2_ref[...] + x2_ref[...]) * INV_DEG, w2_ref[...],
                preferred_element_type=jnp.float32) + b2_ref[...]
    o_ref[...] = scale_ref[...] * jnp.tanh(g)


def _mm_out(s2, x2, w2pad, b2pad, scale):
    full = lambda shape: pl.BlockSpec(shape, lambda i: (0, 0))
    return pl.pallas_call(
        _out_body,
        grid=(NPAD // MM_R,),
        in_specs=[
            pl.BlockSpec((MM_R, H), lambda i: (i, 0)),
            pl.BlockSpec((MM_R, H), lambda i: (i, 0)),
            full((H, FPAD)), full((1, FPAD)), full((1, FPAD)),
        ],
        out_specs=pl.BlockSpec((MM_R, FPAD), lambda i: (i, 0)),
        out_shape=jax.ShapeDtypeStruct((NPAD, FPAD), jnp.float32),
    )(s2, x2, w2pad, b2pad.reshape(1, FPAD), scale.reshape(1, FPAD))


def _gather_sum_body(table_hbm, idx_hbm, out_hbm, idx_v, rows_v, acc_v,
                     gsem0, gsem1, gsem2, gsem3, ssem0, ssem1, ssem2, ssem3):
    c = lax.axis_index("c")
    s = lax.axis_index("s")
    is0 = c == 0
    base = jnp.where(is0, s * C0_ROWS, C1_BASE + s * C1_ROWS)
    npairs = jnp.where(is0, C0_PAIRS, C1_PAIRS)
    cn = CHUNK_NODES * K
    pltpu.sync_copy(idx_hbm.at[pl.ds(base * K, IDX_MAX)], idx_v)

    def g_start(ci, slot, sem):
        pltpu.async_copy(table_hbm.at[idx_v.at[pl.ds(ci * cn, cn)]],
                         rows_v.at[slot], sem)

    def g_wait(slot, sem):
        pltpu.make_async_copy(table_hbm.at[idx_v.at[pl.ds(0, cn)]],
                              rows_v.at[slot], sem).wait()

    def s_start(ci, slot, sem):
        pltpu.async_copy(acc_v.at[slot],
                         out_hbm.at[pl.ds(base + ci * CHUNK_NODES,
                                          CHUNK_NODES), :], sem)

    def s_wait(slot, sem):
        pltpu.make_async_copy(acc_v.at[slot],
                              out_hbm.at[pl.ds(base, CHUNK_NODES), :],
                              sem).wait()

    def accum(slot):
        for nl in range(CHUNK_NODES):
            for v in range(VREGS_PER_ROW):
                vals = [rows_v[slot, nl * K + j, pl.ds(v * 16, 16)]
                        for j in range(K)]
                while len(vals) > 1:
                    vals = [vals[2 * i] + vals[2 * i + 1]
                            for i in range(len(vals) // 2)]
                acc_v[slot, nl, pl.ds(v * 16, 16)] = vals[0]

    nchunks = npairs * 2
    gsems = [gsem0, gsem1, gsem2, gsem3]
    ssems = [ssem0, ssem1, ssem2, ssem3]
    for b in range(NBUF):
        g_start(b, b, gsems[b])

    def quad(q, carry):
        for b in range(NBUF):
            ci = q * NBUF + b
            g_wait(b, gsems[b])

            @pl.when(q > 0)
            def _():
                s_wait(b, ssems[b])

            accum(b)
            s_start(ci, b, ssems[b])

            @pl.when(ci + NBUF < nchunks)
            def _():
                g_start(ci + NBUF, b, gsems[b])
        return carry

    lax.fori_loop(0, npairs * 2 // NBUF, quad, 0)
    for b in range(NBUF):
        s_wait(b, ssems[b])


def _gather_sum(table, idx_flat):
    mesh = plsc.VectorSubcoreMesh(core_axis_name="c", subcore_axis_name="s")
    f = pl.kernel(
        _gather_sum_body,
        out_type=jax.ShapeDtypeStruct((NPAD, H), jnp.float32),
        mesh=mesh,
        scratch_types=[
            pltpu.VMEM((IDX_MAX,), jnp.int32),
            pltpu.VMEM((NBUF, CHUNK_NODES * K, H), jnp.float32),
            pltpu.VMEM((NBUF, CHUNK_NODES, H), jnp.float32),
        ] + [pltpu.SemaphoreType.DMA] * (2 * NBUF),
    )
    return f(table, idx_flat)


def kernel(pos, features, W1, b1, Wm1, bm1, Wm2, bm2, W2, b2):
    idx = _knn_indices(pos)
    idx_flat = jnp.pad(idx.reshape(NTOT, K),
                       ((0, NPAD - NTOT), (0, 0))).reshape(-1)
    idx_flat = jnp.pad(idx_flat, (0, IDX_PAD - NPAD * K))

    x = jnp.concatenate([pos.reshape(NTOT, AXIS),
                         features.reshape(NTOT, NF)], axis=-1)
    xpad = jnp.pad(x, ((0, NPAD - NTOT), (0, FPAD - FIN)))
    w1pad = jnp.pad(W1, ((0, FPAD - FIN), (0, 0)))
    h1 = _mm_h1(xpad, w1pad)

    s1 = _gather_sum(h1, idx_flat)
    x2 = _mm_mlp(s1, h1, b1, Wm1, bm1, Wm2, bm2)
    s2 = _gather_sum(x2, idx_flat)

    w2pad = jnp.pad(W2, ((0, 0), (0, FPAD - FIN)))
    b2pad = jnp.pad(b2, (0, FPAD - FIN))
    scale = jnp.concatenate([
        jnp.full((AXIS,), MAX_DELTA, jnp.float32),
        jnp.full((NF,), 0.1 * MAX_DELTA, jnp.float32),
        jnp.zeros((FPAD - FIN,), jnp.float32),
    ])
    g = _mm_out(s2, x2, w2pad, b2pad, scale)

    delta_pos = g[:NTOT, :AXIS].reshape(B, N, AXIS)
    delta_features = g[:NTOT, AXIS:FIN].reshape(B, N, NF)
    return delta_pos, delta_features


# speedup vs baseline: 1.9484x; 1.9484x over previous
import functools

import jax
import jax.numpy as jnp
from jax import lax
from jax.experimental import pallas as pl
from jax.experimental.pallas import tpu as pltpu
from jax.experimental.pallas import tpu_sc as plsc

B, N, AXIS, NF, K, H = 10, 1000, 3, 128, 16, 128
MAX_DELTA = 0.4
NTOT = B * N
NROW = 1024
NPAD = B * NROW
NCOLPAD = 1024
FIN = AXIS + NF
FPAD = 256
INV_DEG = 1.0 / float(K + 1)

SC_ROWS = 256
TC_ROWS = N - SC_ROWS
SC_TOT = B * SC_ROWS

SC_NC = 2
SC_NS = 16
SC_NW = SC_NC * SC_NS
ROWS_PER_W = SC_TOT // SC_NW
CHUNK_NODES = 8
NCHUNKS = ROWS_PER_W // CHUNK_NODES
NBUF = 2
IDX_MAX = ROWS_PER_W * K
VREGS_PER_ROW = H // 16


KNN_R = 200


def _knn_body(pos_row_ref, pos_col_ref, idx_ref, a_ref):
    b = pl.program_id(0)
    pr = pos_row_ref[0]
    pc = pos_col_ref[0]
    dx = pr[:, 0:1] - pc[0:1, :]
    dy = pr[:, 1:2] - pc[1:2, :]
    dz = pr[:, 2:3] - pc[2:3, :]
    d2 = (dx * dx + dy * dy) + dz * dz
    lane = lax.broadcasted_iota(jnp.int32, (KNN_R, NCOLPAD), 1
                                ).astype(jnp.float32)
    d2 = jnp.where(lane >= float(N), jnp.inf, d2)
    col16 = lax.broadcasted_iota(jnp.int32, (KNN_R, K), 1)
    acc = jnp.zeros((KNN_R, K), jnp.int32)
    abuild = jnp.zeros((KNN_R, NCOLPAD), jnp.float32)
    big = jnp.float32(3e9)
    for t in range(K):
        m = jnp.min(d2, axis=1, keepdims=True)
        cand = jnp.where(d2 == m, lane, big)
        sel = jnp.min(cand, axis=1, keepdims=True)
        acc = jnp.where(col16 == t, sel.astype(jnp.int32), acc)
        hit = lane == sel
        d2 = jnp.where(hit, jnp.inf, d2)
        abuild = jnp.where(hit, 1.0, abuild)
    idx_ref[0] = acc + b * NROW
    a_ref[0] = abuild


def _knn_indices(pos):
    pos_row = jnp.pad(pos, ((0, 0), (0, NCOLPAD - N), (0, 8 - AXIS)))
    pos_col = jnp.pad(jnp.transpose(pos, (0, 2, 1)),
                      ((0, 0), (0, 8 - AXIS), (0, NCOLPAD - N)))
    grid = (B, N // KNN_R)
    return pl.pallas_call(
        _knn_body,
        grid=grid,
        in_specs=[
            pl.BlockSpec((1, KNN_R, 8), lambda b, r: (b, r, 0)),
            pl.BlockSpec((1, 8, NCOLPAD), lambda b, r: (b, 0, 0)),
        ],
        out_specs=[pl.BlockSpec((1, KNN_R, K), lambda b, r: (b, r, 0)),
                   pl.BlockSpec((1, KNN_R, NCOLPAD), lambda b, r: (b, r, 0))],
        out_shape=[jax.ShapeDtypeStruct((B, N, K), jnp.int32),
                   jax.ShapeDtypeStruct((B, N, NCOLPAD), jnp.float32)],
    )(pos_row, pos_col)


MM_R = 512


def _h1_body(x_ref, w_ref, o_ref):
    o_ref[...] = jnp.dot(x_ref[...], w_ref[...],
                         preferred_element_type=jnp.float32)


def _mm_h1(xpad, w1pad):
    return pl.pallas_call(
        _h1_body,
        grid=(NPAD // MM_R,),
        in_specs=[
            pl.BlockSpec((MM_R, FPAD), lambda i: (i, 0)),
            pl.BlockSpec((FPAD, H), lambda i: (0, 0)),
        ],
        out_specs=pl.BlockSpec((MM_R, H), lambda i: (i, 0)),
        out_shape=jax.ShapeDtypeStruct((NPAD, H), jnp.float32),
    )(xpad, w1pad)


def _agg_body(a_ref, h_ref, o_ref):
    o_ref[0] = jnp.dot(a_ref[0], h_ref[0], preferred_element_type=jnp.float32)


def _mm_agg(adj, hB):
    return pl.pallas_call(
        _agg_body,
        grid=(B,),
        in_specs=[
            pl.BlockSpec((1, TC_ROWS, NCOLPAD), lambda b: (b, 0, 0)),
            pl.BlockSpec((1, NROW, H), lambda b: (b, 0, 0)),
        ],
        out_specs=pl.BlockSpec((1, TC_ROWS, H), lambda b: (b, 0, 0)),
        out_shape=jax.ShapeDtypeStruct((B, TC_ROWS, H), jnp.float32),
    )(adj, hB)


def _mlp_body(s1_ref, h1_ref, b1_ref, wm1_ref, bm1_ref, wm2_ref, bm2_ref, o_ref):
    x1 = jax.nn.relu((s1_ref[...] + h1_ref[...]) * INV_DEG + b1_ref[...])
    t = jax.nn.relu(jnp.dot(x1, wm1_ref[...],
                            preferred_element_type=jnp.float32) + bm1_ref[...])
    o_ref[...] = jnp.dot(t, wm2_ref[...],
                         preferred_element_type=jnp.float32) + bm2_ref[...]


def _mm_mlp(s1, h1, b1, wm1, bm1, wm2, bm2):
    full = lambda shape: pl.BlockSpec(shape, lambda i: (0, 0))
    return pl.pallas_call(
        _mlp_body,
        grid=(NPAD // MM_R,),
        in_specs=[
            pl.BlockSpec((MM_R, H), lambda i: (i, 0)),
            pl.BlockSpec((MM_R, H), lambda i: (i, 0)),
            full((1, H)), full((H, H)), full((1, H)), full((H, H)), full((1, H)),
        ],
        out_specs=pl.BlockSpec((MM_R, H), lambda i: (i, 0)),
        out_shape=jax.ShapeDtypeStruct((NPAD, H), jnp.float32),
    )(s1, h1, b1.reshape(1, H), wm1, bm1.reshape(1, H), wm2, bm2.reshape(1, H))


def _out_body(s2_ref, x2_ref, w2_ref, b2_ref, scale_ref, o_ref):
    g = jnp.dot((s2_ref[...] + x2_ref[...]) * INV_DEG, w2_ref[...],
                preferred_element_type=jnp.float32) + b2_ref[...]
    o_ref[...] = scale_ref[...] * jnp.tanh(g)


def _mm_out(s2, x2, w2pad, b2pad, scale):
    full = lambda shape: pl.BlockSpec(shape, lambda i: (0, 0))
    return pl.pallas_call(
        _out_body,
        grid=(NPAD // MM_R,),
        in_specs=[
            pl.BlockSpec((MM_R, H), lambda i: (i, 0)),
            pl.BlockSpec((MM_R, H), lambda i: (i, 0)),
            full((H, FPAD)), full((1, FPAD)), full((1, FPAD)),
        ],
        out_specs=pl.BlockSpec((MM_R, FPAD), lambda i: (i, 0)),
        out_shape=jax.ShapeDtypeStruct((NPAD, FPAD), jnp.float32),
    )(s2, x2, w2pad, b2pad.reshape(1, FPAD), scale.reshape(1, FPAD))


def _gather_sum_body(table_hbm, idx_hbm, out_hbm, idx_v, rows_v, acc_v,
                     gsem0, gsem1, ssem0, ssem1):
    c = lax.axis_index("c")
    s = lax.axis_index("s")
    wid = s * SC_NC + c
    base = wid * ROWS_PER_W
    cn = CHUNK_NODES * K
    pltpu.sync_copy(idx_hbm.at[pl.ds(base * K, IDX_MAX)], idx_v)

    def g_start(ci, slot, sem):
        pltpu.async_copy(table_hbm.at[idx_v.at[pl.ds(ci * cn, cn)]],
                         rows_v.at[slot], sem)

    def g_wait(slot, sem):
        pltpu.make_async_copy(table_hbm.at[idx_v.at[pl.ds(0, cn)]],
                              rows_v.at[slot], sem).wait()

    def s_start(ci, slot, sem):
        pltpu.async_copy(acc_v.at[slot],
                         out_hbm.at[pl.ds(base + ci * CHUNK_NODES,
                                          CHUNK_NODES), :], sem)

    def s_wait(slot, sem):
        pltpu.make_async_copy(acc_v.at[slot],
                              out_hbm.at[pl.ds(base, CHUNK_NODES), :],
                              sem).wait()

    def accum(slot):
        for nl in range(CHUNK_NODES):
            for v in range(VREGS_PER_ROW):
                vals = [rows_v[slot, nl * K + j, pl.ds(v * 16, 16)]
                        for j in range(K)]
                while len(vals) > 1:
                    vals = [vals[2 * i] + vals[2 * i + 1]
                            for i in range(len(vals) // 2)]
                acc_v[slot, nl, pl.ds(v * 16, 16)] = vals[0]

    gsems = [gsem0, gsem1]
    ssems = [ssem0, ssem1]
    for bslot in range(NBUF):
        g_start(bslot, bslot, gsems[bslot])

    def pair(p, carry):
        for bslot in range(NBUF):
            ci = p * NBUF + bslot
            g_wait(bslot, gsems[bslot])

            @pl.when(p > 0)
            def _():
                s_wait(bslot, ssems[bslot])

            accum(bslot)
            s_start(ci, bslot, ssems[bslot])

            @pl.when(ci + NBUF < NCHUNKS)
            def _():
                g_start(ci + NBUF, bslot, gsems[bslot])
        return carry

    lax.fori_loop(0, NCHUNKS // NBUF, pair, 0)
    for bslot in range(NBUF):
        s_wait(bslot, ssems[bslot])


def _gather_sum(table, idx_sc):
    mesh = plsc.VectorSubcoreMesh(core_axis_name="c", subcore_axis_name="s")
    f = pl.kernel(
        _gather_sum_body,
        out_type=jax.ShapeDtypeStruct((SC_TOT, H), jnp.float32),
        mesh=mesh,
        scratch_types=[
            pltpu.VMEM((IDX_MAX,), jnp.int32),
            pltpu.VMEM((NBUF, CHUNK_NODES * K, H), jnp.float32),
            pltpu.VMEM((NBUF, CHUNK_NODES, H), jnp.float32),
        ] + [pltpu.SemaphoreType.DMA] * (2 * NBUF),
    )
    return f(table, idx_sc)


def _aggregate(table, adj, idx_sc):
    s_sc = _gather_sum(table, idx_sc)
    s_tc = _mm_agg(adj, table.reshape(B, NROW, H))
    s = jnp.concatenate([s_tc, s_sc.reshape(B, SC_ROWS, H)], axis=1)
    return jnp.pad(s, ((0, 0), (0, NROW - N), (0, 0))).reshape(NPAD, H)


def kernel(pos, features, W1, b1, Wm1, bm1, Wm2, bm2, W2, b2):
    idx, adj = _knn_indices(pos)
    idx_sc = idx[:, TC_ROWS:, :].reshape(-1)

    x = jnp.concatenate([pos, features], axis=-1)
    xpad = jnp.pad(x, ((0, 0), (0, NROW - N), (0, FPAD - FIN))
                   ).reshape(NPAD, FPAD)
    w1pad = jnp.pad(W1, ((0, FPAD - FIN), (0, 0)))
    h1 = _mm_h1(xpad, w1pad)

    s1 = _aggregate(h1, adj, idx_sc)
    x2 = _mm_mlp(s1, h1, b1, Wm1, bm1, Wm2, bm2)
    s2 = _aggregate(x2, adj, idx_sc)

    w2pad = jnp.pad(W2, ((0, 0), (0, FPAD - FIN)))
    b2pad = jnp.pad(b2, (0, FPAD - FIN))
    scale = jnp.concatenate([
        jnp.full((AXIS,), MAX_DELTA, jnp.float32),
        jnp.full((NF,), 0.1 * MAX_DELTA, jnp.float32),
        jnp.zeros((FPAD - FIN,), jnp.float32),
    ])
    g = _mm_out(s2, x2, w2pad, b2pad, scale)

    gB = g.reshape(B, NROW, FPAD)[:, :N]
    delta_pos = gB[..., :AXIS]
    delta_features = gB[..., AXIS:FIN]
    return delta_pos, delta_features


# speedup vs baseline: 2.0563x; 1.0554x over previous
import functools

import jax
import jax.numpy as jnp
from jax import lax
from jax.experimental import pallas as pl
from jax.experimental.pallas import tpu as pltpu
from jax.experimental.pallas import tpu_sc as plsc

B, N, AXIS, NF, K, H = 10, 1000, 3, 128, 16, 128
MAX_DELTA = 0.4
NTOT = B * N
NROW = 1024
NPAD = B * NROW
NCOLPAD = 1024
FIN = AXIS + NF
FPAD = 256
INV_DEG = 1.0 / float(K + 1)

SC_ROWS = 256
TC_ROWS = N - SC_ROWS
SC_TOT = B * SC_ROWS

SC_NC = 2
SC_NS = 16
SC_NW = SC_NC * SC_NS
ROWS_PER_W = SC_TOT // SC_NW
CHUNK_NODES = 8
NCHUNKS = ROWS_PER_W // CHUNK_NODES
NBUF = 2
IDX_MAX = ROWS_PER_W * K
VREGS_PER_ROW = H // 16


KNN_R = 200


def _knn_body(pos_row_ref, pos_col_ref, idx_ref, a_ref):
    b = pl.program_id(0)
    pr = pos_row_ref[0]
    pc = pos_col_ref[0]
    dx = pr[:, 0:1] - pc[0:1, :]
    dy = pr[:, 1:2] - pc[1:2, :]
    dz = pr[:, 2:3] - pc[2:3, :]
    d2 = (dx * dx + dy * dy) + dz * dz
    lane = lax.broadcasted_iota(jnp.int32, (KNN_R, NCOLPAD), 1
                                ).astype(jnp.float32)
    d2 = jnp.where(lane >= float(N), jnp.inf, d2)
    d2o = d2
    col16 = lax.broadcasted_iota(jnp.int32, (KNN_R, K), 1)
    acc = jnp.zeros((KNN_R, K), jnp.int32)
    big = jnp.float32(3e9)
    m = sel = None
    for t in range(K):
        m = jnp.min(d2, axis=1, keepdims=True)
        cand = jnp.where(d2 == m, lane, big)
        sel = jnp.min(cand, axis=1, keepdims=True)
        acc = jnp.where(col16 == t, sel.astype(jnp.int32), acc)
        d2 = jnp.where(lane == sel, jnp.inf, d2)
    idx_ref[0] = acc + b * NROW
    a_ref[0] = jnp.where((d2o < m) | ((d2o == m) & (lane <= sel)), 1.0, 0.0)


def _knn_indices(pos):
    pos_row = jnp.pad(pos, ((0, 0), (0, NCOLPAD - N), (0, 8 - AXIS)))
    pos_col = jnp.pad(jnp.transpose(pos, (0, 2, 1)),
                      ((0, 0), (0, 8 - AXIS), (0, NCOLPAD - N)))
    grid = (B, N // KNN_R)
    return pl.pallas_call(
        _knn_body,
        grid=grid,
        in_specs=[
            pl.BlockSpec((1, KNN_R, 8), lambda b, r: (b, r, 0)),
            pl.BlockSpec((1, 8, NCOLPAD), lambda b, r: (b, 0, 0)),
        ],
        out_specs=[pl.BlockSpec((1, KNN_R, K), lambda b, r: (b, r, 0)),
                   pl.BlockSpec((1, KNN_R, NCOLPAD), lambda b, r: (b, r, 0))],
        out_shape=[jax.ShapeDtypeStruct((B, N, K), jnp.int32),
                   jax.ShapeDtypeStruct((B, N, NCOLPAD), jnp.float32)],
    )(pos_row, pos_col)


MM_R = 512


def _h1_body(x_ref, w_ref, o_ref):
    o_ref[...] = jnp.dot(x_ref[...], w_ref[...],
                         preferred_element_type=jnp.float32)


def _mm_h1(xpad, w1pad):
    return pl.pallas_call(
        _h1_body,
        grid=(NPAD // MM_R,),
        in_specs=[
            pl.BlockSpec((MM_R, FPAD), lambda i: (i, 0)),
            pl.BlockSpec((FPAD, H), lambda i: (0, 0)),
        ],
        out_specs=pl.BlockSpec((MM_R, H), lambda i: (i, 0)),
        out_shape=jax.ShapeDtypeStruct((NPAD, H), jnp.float32),
    )(xpad, w1pad)


def _agg_body(a_ref, h_ref, o_ref):
    o_ref[0] = jnp.dot(a_ref[0], h_ref[0], preferred_element_type=jnp.float32)


def _mm_agg(adj, hB):
    return pl.pallas_call(
        _agg_body,
        grid=(B,),
        in_specs=[
            pl.BlockSpec((1, TC_ROWS, NCOLPAD), lambda b: (b, 0, 0)),
            pl.BlockSpec((1, NROW, H), lambda b: (b, 0, 0)),
        ],
        out_specs=pl.BlockSpec((1, TC_ROWS, H), lambda b: (b, 0, 0)),
        out_shape=jax.ShapeDtypeStruct((B, TC_ROWS, H), jnp.float32),
    )(adj, hB)


def _mlp_body(s1_ref, h1_ref, b1_ref, wm1_ref, bm1_ref, wm2_ref, bm2_ref, o_ref):
    x1 = jax.nn.relu((s1_ref[...] + h1_ref[...]) * INV_DEG + b1_ref[...])
    t = jax.nn.relu(jnp.dot(x1, wm1_ref[...],
                            preferred_element_type=jnp.float32) + bm1_ref[...])
    o_ref[...] = jnp.dot(t, wm2_ref[...],
                         preferred_element_type=jnp.float32) + bm2_ref[...]


def _mm_mlp(s1, h1, b1, wm1, bm1, wm2, bm2):
    full = lambda shape: pl.BlockSpec(shape, lambda i: (0, 0))
    return pl.pallas_call(
        _mlp_body,
        grid=(NPAD // MM_R,),
        in_specs=[
            pl.BlockSpec((MM_R, H), lambda i: (i, 0)),
            pl.BlockSpec((MM_R, H), lambda i: (i, 0)),
            full((1, H)), full((H, H)), full((1, H)), full((H, H)), full((1, H)),
        ],
        out_specs=pl.BlockSpec((MM_R, H), lambda i: (i, 0)),
        out_shape=jax.ShapeDtypeStruct((NPAD, H), jnp.float32),
    )(s1, h1, b1.reshape(1, H), wm1, bm1.reshape(1, H), wm2, bm2.reshape(1, H))


def _out_body(s2_ref, x2_ref, w2_ref, b2_ref, scale_ref, o_ref):
    g = jnp.dot((s2_ref[...] + x2_ref[...]) * INV_DEG, w2_ref[...],
                preferred_element_type=jnp.float32) + b2_ref[...]
    o_ref[...] = scale_ref[...] * jnp.tanh(g)


def _mm_out(s2, x2, w2pad, b2pad, scale):
    full = lambda shape: pl.BlockSpec(shape, lambda i: (0, 0))
    return pl.pallas_call(
        _out_body,
        grid=(NPAD // MM_R,),
        in_specs=[
            pl.BlockSpec((MM_R, H), lambda i: (i, 0)),
            pl.BlockSpec((MM_R, H), lambda i: (i, 0)),
            full((H, FPAD)), full((1, FPAD)), full((1, FPAD)),
        ],
        out_specs=pl.BlockSpec((MM_R, FPAD), lambda i: (i, 0)),
        out_shape=jax.ShapeDtypeStruct((NPAD, FPAD), jnp.float32),
    )(s2, x2, w2pad, b2pad.reshape(1, FPAD), scale.reshape(1, FPAD))


def _gather_sum_body(table_hbm, idx_hbm, out_hbm, idx_v, rows_v, acc_v,
                     gsem0, gsem1, ssem0, ssem1):
    c = lax.axis_index("c")
    s = lax.axis_index("s")
    wid = s * SC_NC + c
    base = wid * ROWS_PER_W
    cn = CHUNK_NODES * K
    pltpu.sync_copy(idx_hbm.at[pl.ds(base * K, IDX_MAX)], idx_v)

    def g_start(ci, slot, sem):
        pltpu.async_copy(table_hbm.at[idx_v.at[pl.ds(ci * cn, cn)]],
                         rows_v.at[slot], sem)

    def g_wait(slot, sem):
        pltpu.make_async_copy(table_hbm.at[idx_v.at[pl.ds(0, cn)]],
                              rows_v.at[slot], sem).wait()

    def s_start(ci, slot, sem):
        pltpu.async_copy(acc_v.at[slot],
                         out_hbm.at[pl.ds(base + ci * CHUNK_NODES,
                                          CHUNK_NODES), :], sem)

    def s_wait(slot, sem):
        pltpu.make_async_copy(acc_v.at[slot],
                              out_hbm.at[pl.ds(base, CHUNK_NODES), :],
                              sem).wait()

    def accum(slot):
        for nl in range(CHUNK_NODES):
            for v in range(VREGS_PER_ROW):
                vals = [rows_v[slot, nl * K + j, pl.ds(v * 16, 16)]
                        for j in range(K)]
                while len(vals) > 1:
                    vals = [vals[2 * i] + vals[2 * i + 1]
                            for i in range(len(vals) // 2)]
                acc_v[slot, nl, pl.ds(v * 16, 16)] = vals[0]

    gsems = [gsem0, gsem1]
    ssems = [ssem0, ssem1]
    for bslot in range(NBUF):
        g_start(bslot, bslot, gsems[bslot])

    def pair(p, carry):
        for bslot in range(NBUF):
            ci = p * NBUF + bslot
            g_wait(bslot, gsems[bslot])

            @pl.when(p > 0)
            def _():
                s_wait(bslot, ssems[bslot])

            accum(bslot)
            s_start(ci, bslot, ssems[bslot])

            @pl.when(ci + NBUF < NCHUNKS)
            def _():
                g_start(ci + NBUF, bslot, gsems[bslot])
        return carry

    lax.fori_loop(0, NCHUNKS // NBUF, pair, 0)
    for bslot in range(NBUF):
        s_wait(bslot, ssems[bslot])


def _gather_sum(table, idx_sc):
    mesh = plsc.VectorSubcoreMesh(core_axis_name="c", subcore_axis_name="s")
    f = pl.kernel(
        _gather_sum_body,
        out_type=jax.ShapeDtypeStruct((SC_TOT, H), jnp.float32),
        mesh=mesh,
        scratch_types=[
            pltpu.VMEM((IDX_MAX,), jnp.int32),
            pltpu.VMEM((NBUF, CHUNK_NODES * K, H), jnp.float32),
            pltpu.VMEM((NBUF, CHUNK_NODES, H), jnp.float32),
        ] + [pltpu.SemaphoreType.DMA] * (2 * NBUF),
    )
    return f(table, idx_sc)


def _aggregate(table, adj, idx_sc):
    s_sc = _gather_sum(table, idx_sc)
    s_tc = _mm_agg(adj, table.reshape(B, NROW, H))
    s = jnp.concatenate([s_tc, s_sc.reshape(B, SC_ROWS, H)], axis=1)
    return jnp.pad(s, ((0, 0), (0, NROW - N), (0, 0))).reshape(NPAD, H)


def kernel(pos, features, W1, b1, Wm1, bm1, Wm2, bm2, W2, b2):
    idx, adj = _knn_indices(pos)
    idx_sc = idx[:, TC_ROWS:, :].reshape(-1)

    x = jnp.concatenate([pos, features], axis=-1)
    xpad = jnp.pad(x, ((0, 0), (0, NROW - N), (0, FPAD - FIN))
                   ).reshape(NPAD, FPAD)
    w1pad = jnp.pad(W1, ((0, FPAD - FIN), (0, 0)))
    h1 = _mm_h1(xpad, w1pad)

    s1 = _aggregate(h1, adj, idx_sc)
    x2 = _mm_mlp(s1, h1, b1, Wm1, bm1, Wm2, bm2)
    s2 = _aggregate(x2, adj, idx_sc)

    w2pad = jnp.pad(W2, ((0, 0), (0, FPAD - FIN)))
    b2pad = jnp.pad(b2, (0, FPAD - FIN))
    scale = jnp.concatenate([
        jnp.full((AXIS,), MAX_DELTA, jnp.float32),
        jnp.full((NF,), 0.1 * MAX_DELTA, jnp.float32),
        jnp.zeros((FPAD - FIN,), jnp.float32),
    ])
    g = _mm_out(s2, x2, w2pad, b2pad, scale)

    gB = g.reshape(B, NROW, FPAD)[:, :N]
    delta_pos = gB[..., :AXIS]
    delta_features = gB[..., AXIS:FIN]
    return delta_pos, delta_features


# speedup vs baseline: 2.2812x; 1.1094x over previous
import functools

import jax
import jax.numpy as jnp
from jax import lax
from jax.experimental import pallas as pl
from jax.experimental.pallas import tpu as pltpu
from jax.experimental.pallas import tpu_sc as plsc

B, N, AXIS, NF, K, H = 10, 1000, 3, 128, 16, 128
MAX_DELTA = 0.4
NTOT = B * N
NROW = 1024
NPAD = B * NROW
NCOLPAD = 1024
FIN = AXIS + NF
FPAD = 256
INV_DEG = 1.0 / float(K + 1)

SC_ROWS = 128
TC_ROWS = N - SC_ROWS
SC_TOT = B * SC_ROWS

SC_NC = 2
SC_NS = 16
SC_NW = SC_NC * SC_NS
ROWS_PER_W = SC_TOT // SC_NW
CHUNK_NODES = 4
NCHUNKS = ROWS_PER_W // CHUNK_NODES
NBUF = 2
IDX_MAX = ROWS_PER_W * K
VREGS_PER_ROW = H // 16


KNN_R = 200


def _knn_body(pos_row_ref, pos_col_ref, idx_ref, a_ref):
    b = pl.program_id(0)
    pr = pos_row_ref[0]
    pc = pos_col_ref[0]
    dx = pr[:, 0:1] - pc[0:1, :]
    dy = pr[:, 1:2] - pc[1:2, :]
    dz = pr[:, 2:3] - pc[2:3, :]
    d2 = (dx * dx + dy * dy) + dz * dz
    lane = lax.broadcasted_iota(jnp.int32, (KNN_R, NCOLPAD), 1
                                ).astype(jnp.float32)
    d2 = jnp.where(lane >= float(N), jnp.inf, d2)
    d2o = d2
    col16 = lax.broadcasted_iota(jnp.int32, (KNN_R, K), 1)
    acc = jnp.zeros((KNN_R, K), jnp.int32)
    big = jnp.float32(3e9)
    m = sel = None
    for t in range(K):
        m = jnp.min(d2, axis=1, keepdims=True)
        cand = jnp.where(d2 == m, lane, big)
        sel = jnp.min(cand, axis=1, keepdims=True)
        acc = jnp.where(col16 == t, sel.astype(jnp.int32), acc)
        d2 = jnp.where(lane == sel, jnp.inf, d2)
    idx_ref[0] = acc + b * NROW
    a_ref[0] = jnp.where((d2o < m) | ((d2o == m) & (lane <= sel)), 1.0, 0.0)


def _knn_indices(pos):
    pos_row = jnp.pad(pos, ((0, 0), (0, NCOLPAD - N), (0, 8 - AXIS)))
    pos_col = jnp.pad(jnp.transpose(pos, (0, 2, 1)),
                      ((0, 0), (0, 8 - AXIS), (0, NCOLPAD - N)))
    grid = (B, N // KNN_R)
    return pl.pallas_call(
        _knn_body,
        grid=grid,
        in_specs=[
            pl.BlockSpec((1, KNN_R, 8), lambda b, r: (b, r, 0)),
            pl.BlockSpec((1, 8, NCOLPAD), lambda b, r: (b, 0, 0)),
        ],
        out_specs=[pl.BlockSpec((1, KNN_R, K), lambda b, r: (b, r, 0)),
                   pl.BlockSpec((1, KNN_R, NCOLPAD), lambda b, r: (b, r, 0))],
        out_shape=[jax.ShapeDtypeStruct((B, N, K), jnp.int32),
                   jax.ShapeDtypeStruct((B, N, NCOLPAD), jnp.float32)],
    )(pos_row, pos_col)


MM_R = 512


def _h1_body(x_ref, w_ref, o_ref):
    o_ref[...] = jnp.dot(x_ref[...], w_ref[...],
                         preferred_element_type=jnp.float32)


def _mm_h1(xpad, w1pad):
    return pl.pallas_call(
        _h1_body,
        grid=(NPAD // MM_R,),
        in_specs=[
            pl.BlockSpec((MM_R, FPAD), lambda i: (i, 0)),
            pl.BlockSpec((FPAD, H), lambda i: (0, 0)),
        ],
        out_specs=pl.BlockSpec((MM_R, H), lambda i: (i, 0)),
        out_shape=jax.ShapeDtypeStruct((NPAD, H), jnp.float32),
    )(xpad, w1pad)


def _agg_body(a_ref, h_ref, o_ref):
    o_ref[0] = jnp.dot(a_ref[0], h_ref[0], preferred_element_type=jnp.float32)


def _mm_agg(adj, hB):
    return pl.pallas_call(
        _agg_body,
        grid=(B,),
        in_specs=[
            pl.BlockSpec((1, TC_ROWS, NCOLPAD), lambda b: (b, 0, 0)),
            pl.BlockSpec((1, NROW, H), lambda b: (b, 0, 0)),
        ],
        out_specs=pl.BlockSpec((1, TC_ROWS, H), lambda b: (b, 0, 0)),
        out_shape=jax.ShapeDtypeStruct((B, TC_ROWS, H), jnp.float32),
    )(adj, hB)


def _mlp_body(s1_ref, h1_ref, b1_ref, wm1_ref, bm1_ref, wm2_ref, bm2_ref, o_ref):
    x1 = jax.nn.relu((s1_ref[...] + h1_ref[...]) * INV_DEG + b1_ref[...])
    t = jax.nn.relu(jnp.dot(x1, wm1_ref[...],
                            preferred_element_type=jnp.float32) + bm1_ref[...])
    o_ref[...] = jnp.dot(t, wm2_ref[...],
                         preferred_element_type=jnp.float32) + bm2_ref[...]


def _mm_mlp(s1, h1, b1, wm1, bm1, wm2, bm2):
    full = lambda shape: pl.BlockSpec(shape, lambda i: (0, 0))
    return pl.pallas_call(
        _mlp_body,
        grid=(NPAD // MM_R,),
        in_specs=[
            pl.BlockSpec((MM_R, H), lambda i: (i, 0)),
            pl.BlockSpec((MM_R, H), lambda i: (i, 0)),
            full((1, H)), full((H, H)), full((1, H)), full((H, H)), full((1, H)),
        ],
        out_specs=pl.BlockSpec((MM_R, H), lambda i: (i, 0)),
        out_shape=jax.ShapeDtypeStruct((NPAD, H), jnp.float32),
    )(s1, h1, b1.reshape(1, H), wm1, bm1.reshape(1, H), wm2, bm2.reshape(1, H))


def _out_body(s2_ref, x2_ref, w2_ref, b2_ref, scale_ref, o_ref):
    g = jnp.dot((s2_ref[...] + x2_ref[...]) * INV_DEG, w2_ref[...],
                preferred_element_type=jnp.float32) + b2_ref[...]
    o_ref[...] = scale_ref[...] * jnp.tanh(g)


def _mm_out(s2, x2, w2pad, b2pad, scale):
    full = lambda shape: pl.BlockSpec(shape, lambda i: (0, 0))
    return pl.pallas_call(
        _out_body,
        grid=(NPAD // MM_R,),
        in_specs=[
            pl.BlockSpec((MM_R, H), lambda i: (i, 0)),
            pl.BlockSpec((MM_R, H), lambda i: (i, 0)),
            full((H, FPAD)), full((1, FPAD)), full((1, FPAD)),
        ],
        out_specs=pl.BlockSpec((MM_R, FPAD), lambda i: (i, 0)),
        out_shape=jax.ShapeDtypeStruct((NPAD, FPAD), jnp.float32),
    )(s2, x2, w2pad, b2pad.reshape(1, FPAD), scale.reshape(1, FPAD))


def _gather_sum_body(table_hbm, idx_hbm, out_hbm, idx_v, rows_v, acc_v,
                     gsem0, gsem1, ssem0, ssem1):
    c = lax.axis_index("c")
    s = lax.axis_index("s")
    wid = s * SC_NC + c
    base = wid * ROWS_PER_W
    cn = CHUNK_NODES * K
    pltpu.sync_copy(idx_hbm.at[pl.ds(base * K, IDX_MAX)], idx_v)

    def g_start(ci, slot, sem):
        pltpu.async_copy(table_hbm.at[idx_v.at[pl.ds(ci * cn, cn)]],
                         rows_v.at[slot], sem)

    def g_wait(slot, sem):
        pltpu.make_async_copy(table_hbm.at[idx_v.at[pl.ds(0, cn)]],
                              rows_v.at[slot], sem).wait()

    def s_start(ci, slot, sem):
        pltpu.async_copy(acc_v.at[slot],
                         out_hbm.at[pl.ds(base + ci * CHUNK_NODES,
                                          CHUNK_NODES), :], sem)

    def s_wait(slot, sem):
        pltpu.make_async_copy(acc_v.at[slot],
                              out_hbm.at[pl.ds(base, CHUNK_NODES), :],
                              sem).wait()

    def accum(slot):
        for nl in range(CHUNK_NODES):
            for v in range(VREGS_PER_ROW):
                vals = [rows_v[slot, nl * K + j, pl.ds(v * 16, 16)]
                        for j in range(K)]
                while len(vals) > 1:
                    vals = [vals[2 * i] + vals[2 * i + 1]
                            for i in range(len(vals) // 2)]
                acc_v[slot, nl, pl.ds(v * 16, 16)] = vals[0]

    gsems = [gsem0, gsem1]
    ssems = [ssem0, ssem1]
    for bslot in range(NBUF):
        g_start(bslot, bslot, gsems[bslot])

    def pair(p, carry):
        for bslot in range(NBUF):
            ci = p * NBUF + bslot
            g_wait(bslot, gsems[bslot])

            @pl.when(p > 0)
            def _():
                s_wait(bslot, ssems[bslot])

            accum(bslot)
            s_start(ci, bslot, ssems[bslot])

            @pl.when(ci + NBUF < NCHUNKS)
            def _():
                g_start(ci + NBUF, bslot, gsems[bslot])
        return carry

    lax.fori_loop(0, NCHUNKS // NBUF, pair, 0)
    for bslot in range(NBUF):
        s_wait(bslot, ssems[bslot])


def _gather_sum(table, idx_sc):
    mesh = plsc.VectorSubcoreMesh(core_axis_name="c", subcore_axis_name="s")
    f = pl.kernel(
        _gather_sum_body,
        out_type=jax.ShapeDtypeStruct((SC_TOT, H), jnp.float32),
        mesh=mesh,
        scratch_types=[
            pltpu.VMEM((IDX_MAX,), jnp.int32),
            pltpu.VMEM((NBUF, CHUNK_NODES * K, H), jnp.float32),
            pltpu.VMEM((NBUF, CHUNK_NODES, H), jnp.float32),
        ] + [pltpu.SemaphoreType.DMA] * (2 * NBUF),
    )
    return f(table, idx_sc)


def _aggregate(table, adj, idx_sc):
    s_sc = _gather_sum(table, idx_sc)
    s_tc = _mm_agg(adj, table.reshape(B, NROW, H))
    s = jnp.concatenate([s_tc, s_sc.reshape(B, SC_ROWS, H)], axis=1)
    return jnp.pad(s, ((0, 0), (0, NROW - N), (0, 0))).reshape(NPAD, H)


def kernel(pos, features, W1, b1, Wm1, bm1, Wm2, bm2, W2, b2):
    idx, adj = _knn_indices(pos)
    idx_sc = idx[:, TC_ROWS:, :].reshape(-1)

    x = jnp.concatenate([pos, features], axis=-1)
    xpad = jnp.pad(x, ((0, 0), (0, NROW - N), (0, FPAD - FIN))
                   ).reshape(NPAD, FPAD)
    w1pad = jnp.pad(W1, ((0, FPAD - FIN), (0, 0)))
    h1 = _mm_h1(xpad, w1pad)

    s1 = _aggregate(h1, adj, idx_sc)
    x2 = _mm_mlp(s1, h1, b1, Wm1, bm1, Wm2, bm2)
    s2 = _aggregate(x2, adj, idx_sc)

    w2pad = jnp.pad(W2, ((0, 0), (0, FPAD - FIN)))
    b2pad = jnp.pad(b2, (0, FPAD - FIN))
    scale = jnp.concatenate([
        jnp.full((AXIS,), MAX_DELTA, jnp.float32),
        jnp.full((NF,), 0.1 * MAX_DELTA, jnp.float32),
        jnp.zeros((FPAD - FIN,), jnp.float32),
    ])
    g = _mm_out(s2, x2, w2pad, b2pad, scale)

    gB = g.reshape(B, NROW, FPAD)[:, :N]
    delta_pos = gB[..., :AXIS]
    delta_features = gB[..., AXIS:FIN]
    return delta_pos, delta_features
